# baseline (speedup 1.0000x reference)
import functools

import jax
import jax.numpy as jnp
from jax import lax
from jax.experimental import pallas as pl
from jax.experimental.pallas import tpu as pltpu
from jax.experimental.pallas import tpu_sc as plsc

_ENT_NUM = 1000000
_REL_NUM = 1000
_DIM = 64
_MARGIN = 1.0
_C = 1.0
_EPS = 0.001
_PD_EPS = 1e-6

_NC = 2
_NS = 16
_NW = _NC * _NS
_B = 32768
_BPW = _B // _NW
_CH = 128
_NCH = _BPW // _CH
_LN = 16


def _sc_scores(heads, rels, tails, w_ent, w_rel, w_hyp):
    mesh = plsc.VectorSubcoreMesh(core_axis_name="c", subcore_axis_name="s")

    @functools.partial(
        pl.kernel,
        mesh=mesh,
        compiler_params=pltpu.CompilerParams(
            use_tc_tiling_on_sc=False, needs_layout_passes=False),
        out_type=jax.ShapeDtypeStruct((_B,), jnp.float32),
        scratch_types=[
            pltpu.VMEM((_BPW,), jnp.int32),
            pltpu.VMEM((_BPW,), jnp.int32),
            pltpu.VMEM((_BPW,), jnp.int32),
            pltpu.VMEM((_CH, _DIM), jnp.float32),
            pltpu.VMEM((_CH, _DIM), jnp.float32),
            pltpu.VMEM((_CH, _DIM), jnp.float32),
            pltpu.VMEM((_CH, _DIM), jnp.float32),
            pltpu.VMEM((_BPW,), jnp.float32),
            pltpu.SemaphoreType.DMA,
        ],
    )
    def body(heads_hbm, rels_hbm, tails_hbm, ent_hbm, rel_hbm, hyp_hbm,
             out_hbm, hv, rv, tv, hbuf, tbuf, rbuf, wbuf, sqv, sem):
        wid = lax.axis_index("s") * _NC + lax.axis_index("c")
        base = wid * _BPW
        pltpu.sync_copy(heads_hbm.at[pl.ds(base, _BPW)], hv)
        pltpu.sync_copy(rels_hbm.at[pl.ds(base, _BPW)], rv)
        pltpu.sync_copy(tails_hbm.at[pl.ds(base, _BPW)], tv)

        for c in range(_NCH):
            cp1 = pltpu.async_copy(ent_hbm.at[hv.at[pl.ds(c * _CH, _CH)]],
                                   hbuf, sem)
            cp2 = pltpu.async_copy(ent_hbm.at[tv.at[pl.ds(c * _CH, _CH)]],
                                   tbuf, sem)
            cp3 = pltpu.async_copy(rel_hbm.at[rv.at[pl.ds(c * _CH, _CH)]],
                                   rbuf, sem)
            cp4 = pltpu.async_copy(hyp_hbm.at[rv.at[pl.ds(c * _CH, _CH)]],
                                   wbuf, sem)
            cp1.wait()
            cp2.wait()
            cp3.wait()
            cp4.wait()

            def group(g, _, c=c):
                rows = g * _LN + lax.iota(jnp.int32, _LN)
                zero = jnp.zeros((_LN,), jnp.float32)

                def dims(d, carry):
                    hw, tw, uu, uw, ww = carry
                    dcol = jnp.full((_LN,), 0, jnp.int32) + d
                    h = plsc.load_gather(hbuf, [rows, dcol])
                    t = plsc.load_gather(tbuf, [rows, dcol])
                    r = plsc.load_gather(rbuf, [rows, dcol])
                    w = plsc.load_gather(wbuf, [rows, dcol])
                    u = h + r - t + _PD_EPS
                    return (hw + h * w, tw + t * w, uu + u * u,
                            uw + u * w, ww + w * w)

                hw, tw, uu, uw, ww = lax.fori_loop(
                    0, _DIM, dims, (zero, zero, zero, zero, zero))
                s = hw - tw
                sq = uu - 2.0 * s * uw + s * s * ww
                sqv[pl.ds(c * _CH + g * _LN, _LN)] = sq
                return 0

            lax.fori_loop(0, _CH // _LN, group, 0)

        pltpu.sync_copy(sqv, out_hbm.at[pl.ds(base, _BPW)])

    return body(heads, rels, tails, w_ent, w_rel, w_hyp)


_EBLK = 8000
_EGRID = _ENT_NUM // _EBLK


def _tc_table_losses(w_ent, w_rel, w_hyp):
    def body(ent_ref, rel_ref, hyp_ref, ent_out, hyp_out, acc_ref):
        i = pl.program_id(0)

        @pl.when(i == 0)
        def _init():
            rel = rel_ref[...]
            hyp = hyp_ref[...]
            rn = jnp.sqrt(jnp.sum(rel * rel, axis=1))
            dots = jnp.sum(hyp * rel, axis=1)
            hyp_out[0, 0] = jnp.sum(jnp.maximum(dots / rn - _EPS * _EPS, 0.0))
            acc_ref[0] = 0.0

        x = ent_ref[...]
        nrm = jnp.sqrt(jnp.sum(x * x, axis=1))
        acc_ref[0] += jnp.sum(jnp.maximum(nrm - 1.0, 0.0))

        @pl.when(i == _EGRID - 1)
        def _fin():
            ent_out[0, 0] = acc_ref[0]

    ent_out, hyp_out = pl.pallas_call(
        body,
        grid=(_EGRID,),
        in_specs=[
            pl.BlockSpec((_EBLK, _DIM), lambda i: (i, 0)),
            pl.BlockSpec((_REL_NUM, _DIM), lambda i: (0, 0)),
            pl.BlockSpec((_REL_NUM, _DIM), lambda i: (0, 0)),
        ],
        out_specs=[
            pl.BlockSpec(memory_space=pltpu.SMEM),
            pl.BlockSpec(memory_space=pltpu.SMEM),
        ],
        out_shape=[
            jax.ShapeDtypeStruct((1, 1), jnp.float32),
            jax.ShapeDtypeStruct((1, 1), jnp.float32),
        ],
        scratch_shapes=[pltpu.SMEM((1,), jnp.float32)],
    )(w_ent, w_rel, w_hyp)
    return ent_out, hyp_out


def _tc_finalize(pos_sq, neg_sq, ent_sum, hyp_sum):
    def body(p_ref, n_ref, e_ref, h_ref, out_ref):
        ps = jnp.sqrt(p_ref[...])
        ns = jnp.sqrt(n_ref[...])
        margin = jnp.sum(jnp.maximum(ps - ns + _MARGIN, 0.0))
        out_ref[0, 0] = (margin / (_B // 2)
                         + _C * (e_ref[0, 0] / _ENT_NUM
                                 + h_ref[0, 0] / _REL_NUM))

    out = pl.pallas_call(
        body,
        in_specs=[
            pl.BlockSpec(memory_space=pltpu.VMEM),
            pl.BlockSpec(memory_space=pltpu.VMEM),
            pl.BlockSpec(memory_space=pltpu.SMEM),
            pl.BlockSpec(memory_space=pltpu.SMEM),
        ],
        out_specs=pl.BlockSpec(memory_space=pltpu.SMEM),
        out_shape=jax.ShapeDtypeStruct((1, 1), jnp.float32),
    )(pos_sq, neg_sq, ent_sum, hyp_sum)
    return out


def kernel(pos_x, neg_x, W_ent, W_rel, W_hyp):
    idx = jnp.concatenate([pos_x, neg_x], axis=0).astype(jnp.int32)
    heads = idx[:, 0]
    rels = idx[:, 1]
    tails = idx[:, 2]

    sq = _sc_scores(heads, rels, tails, W_ent, W_rel, W_hyp)
    ent_sum, hyp_sum = _tc_table_losses(W_ent, W_rel, W_hyp)

    half = _B // 2
    pos_sq = sq[:half].reshape(128, 128)
    neg_sq = sq[half:].reshape(128, 128)
    out = _tc_finalize(pos_sq, neg_sq, ent_sum, hyp_sum)
    return out[0, 0].astype(jnp.float32)
